# baseline (speedup 1.0000x reference)
import functools

import jax
import jax.numpy as jnp
from jax import lax
from jax.experimental import pallas as pl
from jax.experimental.pallas import tpu as pltpu
from jax.experimental.pallas import tpu_sc as plsc

_BS = 16384
_EMB = 128
_R_VOCAB = 1000
_R_PAD = 1024
_CHUNK = 128
_NC = 2
_NS = 16
_NW = _NC * _NS
_NROWS = _BS // _CHUNK
_CPW = _NROWS // _NW
_NTASK = 2 * _CPW
_NBUF = 6
_LOOK = 3
_TCB = 256

_mesh = plsc.VectorSubcoreMesh(core_axis_name="c", subcore_axis_name="s")


@functools.partial(
    pl.kernel,
    mesh=_mesh,
    out_type=(
        jax.ShapeDtypeStruct((_BS, _EMB), jnp.float32),
        jax.ShapeDtypeStruct((_BS, _EMB), jnp.float32),
    ),
    scratch_types=(
        [pltpu.VMEM((_NTASK, _CHUNK), jnp.int32),
         pltpu.VMEM((_NBUF, _CHUNK, _EMB), jnp.float32)]
        + [pltpu.SemaphoreType.DMA] * (2 * _NBUF)
    ),
)
def _lookup_ht(h_hbm, t_hbm, emb_e_hbm, out_h, out_t, idx_v, rows_v, *sems):
    gsem, ssem = sems[:_NBUF], sems[_NBUF:]
    wid = lax.axis_index("s") * _NC + lax.axis_index("c")
    c0 = wid * _CPW

    pltpu.sync_copy(h_hbm.at[pl.ds(c0, _CPW)], idx_v.at[pl.ds(0, _CPW)])
    pltpu.sync_copy(t_hbm.at[pl.ds(c0, _CPW)], idx_v.at[pl.ds(_CPW, _CPW)])

    tasks = []
    for j in range(_CPW):
        tasks.append((j, out_h, (c0 + j) * _CHUNK))
        tasks.append((_CPW + j, out_t, (c0 + j) * _CHUNK))

    def fire_gather(i):
        slot, _, _ = tasks[i]
        return pltpu.async_copy(
            emb_e_hbm.at[idx_v.at[slot]], rows_v.at[i % _NBUF],
            gsem[i % _NBUF])

    g_desc = [None] * _NTASK
    s_desc = [None] * _NTASK
    for i in range(_LOOK):
        g_desc[i] = fire_gather(i)
    for i in range(_NTASK):
        j = i + _LOOK
        if j < _NTASK:
            if j >= _NBUF:
                s_desc[j - _NBUF].wait()
            g_desc[j] = fire_gather(j)
        _, out, obase = tasks[i]
        g_desc[i].wait()
        s_desc[i] = pltpu.async_copy(
            rows_v.at[i % _NBUF], out.at[pl.ds(obase, _CHUNK)], ssem[i % _NBUF])
    for i in range(_NTASK - _NBUF, _NTASK):
        s_desc[i].wait()


def _r_body(idx_ref, hilo_ref, out_ref):
    ids = lax.broadcasted_iota(jnp.int32, (_TCB, 2 * _R_PAD), 1)
    sel = idx_ref[0, 0, :][:, None]
    twohot = ((ids == sel) | (ids == sel + _R_PAD)).astype(jnp.bfloat16)
    out_ref[...] = jnp.dot(
        twohot, hilo_ref[...], preferred_element_type=jnp.float32)


_r_lookup_tc = pl.pallas_call(
    _r_body,
    grid=(_BS // _TCB,),
    in_specs=[
        pl.BlockSpec((1, 1, _TCB), lambda i: (i, 0, 0)),
        pl.BlockSpec((2 * _R_PAD, _EMB), lambda i: (0, 0)),
    ],
    out_specs=pl.BlockSpec((_TCB, _EMB), lambda i: (i, 0)),
    out_shape=jax.ShapeDtypeStruct((_BS, _EMB), jnp.float32),
)


def kernel(x, emb_e, emb_r):
    h = x[:, 0].reshape(_NROWS, _CHUNK)
    t = x[:, 2].reshape(_NROWS, _CHUNK)
    r3 = x[:, 1].reshape(_BS // _TCB, 1, _TCB)
    hi = emb_r.astype(jnp.bfloat16)
    lo = (emb_r - hi.astype(jnp.float32)).astype(jnp.bfloat16)
    pad = ((0, _R_PAD - _R_VOCAB), (0, 0))
    hilo = jnp.concatenate([jnp.pad(hi, pad), jnp.pad(lo, pad)], axis=0)
    r_emb = _r_lookup_tc(r3, hilo)
    h_emb, t_emb = _lookup_ht(h, t, emb_e)
    return (h_emb, r_emb, t_emb)


# speedup vs baseline: 1.3819x; 1.3819x over previous
import functools

import jax
import jax.numpy as jnp
from jax import lax
from jax.experimental import pallas as pl
from jax.experimental.pallas import tpu as pltpu
from jax.experimental.pallas import tpu_sc as plsc

_BS = 16384
_EMB = 128
_R_VOCAB = 1000
_CHUNK = 128
_NC = 2
_NS = 16
_NW = _NC * _NS
_NROWS = _BS // _CHUNK
_CPW = _NROWS // _NW
_BPW = _CPW * _CHUNK
_NTASK = 3 * _CPW
_NBUF = 6
_LOOK = 3
_L = 16

_mesh = plsc.VectorSubcoreMesh(core_axis_name="c", subcore_axis_name="s")

_GDN = lax.GatherDimensionNumbers(
    offset_dims=(), collapsed_slice_dims=(0,), start_index_map=(0,))


def _vgather(v, perm):
    return lax.gather(v, perm[:, None], _GDN, (1,),
                      mode=lax.GatherScatterMode.PROMISE_IN_BOUNDS)


@functools.partial(
    pl.kernel,
    mesh=_mesh,
    out_type=(
        jax.ShapeDtypeStruct((_BS, _EMB), jnp.float32),
        jax.ShapeDtypeStruct((_BS, _EMB), jnp.float32),
        jax.ShapeDtypeStruct((_BS, _EMB), jnp.float32),
    ),
    scratch_types=(
        [pltpu.VMEM((3 * _BPW,), jnp.int32),
         pltpu.VMEM((_NTASK * _CHUNK,), jnp.int32),
         pltpu.VMEM((_NBUF, _CHUNK, _EMB), jnp.float32),
         pltpu.VMEM_SHARED((_R_VOCAB, _EMB), jnp.float32)]
        + [pltpu.SemaphoreType.DMA] * (2 * _NBUF)
    ),
)
def _lookup(x_hbm, emb_e_hbm, emb_r_hbm, out_h, out_r, out_t,
            x_v, idx_v, rows_v, emb_r_sp, *sems):
    gsem, ssem = sems[:_NBUF], sems[_NBUF:]
    wid = lax.axis_index("s") * _NC + lax.axis_index("c")
    c0 = wid * _CPW
    b0 = wid * _BPW

    @pl.when(lax.axis_index("s") == 0)
    def _stage():
        pltpu.sync_copy(emb_r_hbm, emb_r_sp)

    pltpu.sync_copy(x_hbm.at[pl.ds(3 * b0, 3 * _BPW)], x_v)
    iota = lax.broadcasted_iota(jnp.int32, (_L,), 0)
    for s in range(3):
        pos = iota * 3 + s
        perm = lax.rem(pos, jnp.full((_L,), _L, jnp.int32))
        m_a = pos < _L
        m_b = pos < 2 * _L
        for g in range(_BPW // _L):
            a = x_v[pl.ds(48 * g, _L)]
            b = x_v[pl.ds(48 * g + _L, _L)]
            c = x_v[pl.ds(48 * g + 2 * _L, _L)]
            ga, gb, gc = (_vgather(v, perm) for v in (a, b, c))
            vals = jnp.where(m_a, ga, jnp.where(m_b, gb, gc))
            idx_v[pl.ds(s * _BPW + g * _L, _L)] = vals

    plsc.subcore_barrier()

    tasks = []
    for j in range(_CPW):
        tasks.append((j, emb_e_hbm, out_h, (c0 + j) * _CHUNK))
        tasks.append((_CPW + j, emb_r_sp, out_r, (c0 + j) * _CHUNK))
        tasks.append((2 * _CPW + j, emb_e_hbm, out_t, (c0 + j) * _CHUNK))

    def fire_gather(i):
        slot, table, _, _ = tasks[i]
        return pltpu.async_copy(
            table.at[idx_v.at[pl.ds(slot * _CHUNK, _CHUNK)]],
            rows_v.at[i % _NBUF], gsem[i % _NBUF])

    g_desc = [None] * _NTASK
    s_desc = [None] * _NTASK
    for i in range(_LOOK):
        g_desc[i] = fire_gather(i)
    for i in range(_NTASK):
        j = i + _LOOK
        if j < _NTASK:
            if j >= _NBUF:
                s_desc[j - _NBUF].wait()
            g_desc[j] = fire_gather(j)
        _, _, out, obase = tasks[i]
        g_desc[i].wait()
        s_desc[i] = pltpu.async_copy(
            rows_v.at[i % _NBUF], out.at[pl.ds(obase, _CHUNK)], ssem[i % _NBUF])
    for i in range(_NTASK - _NBUF, _NTASK):
        s_desc[i].wait()


def kernel(x, emb_e, emb_r):
    return _lookup(x.reshape(-1), emb_e, emb_r)


# speedup vs baseline: 1.6837x; 1.2184x over previous
import functools

import jax
import jax.numpy as jnp
from jax import lax
from jax.experimental import pallas as pl
from jax.experimental.pallas import tpu as pltpu
from jax.experimental.pallas import tpu_sc as plsc

_BS = 16384
_EMB = 128
_R_VOCAB = 1000
_CHUNK = 128
_NC = 2
_NS = 16
_NW = _NC * _NS
_NROWS = _BS // _CHUNK
_CPW = _NROWS // _NW
_BPW = _CPW * _CHUNK
_NTASK = 3 * _CPW
_NBUF = 3
_LOOK = 2
_RSTG = 4

_mesh = plsc.VectorSubcoreMesh(core_axis_name="c", subcore_axis_name="s")


@functools.partial(
    pl.kernel,
    mesh=_mesh,
    out_type=(
        jax.ShapeDtypeStruct((_BS, _EMB), jnp.float32),
        jax.ShapeDtypeStruct((_BS, _EMB), jnp.float32),
        jax.ShapeDtypeStruct((_BS, _EMB), jnp.float32),
    ),
    scratch_types=(
        [pltpu.VMEM((_NTASK, _CHUNK), jnp.int32),
         pltpu.VMEM((_NBUF, _CHUNK, _EMB), jnp.float32),
         pltpu.VMEM_SHARED((_R_VOCAB, _EMB), jnp.float32),
         pltpu.VMEM_SHARED((_NS * _RSTG * _CHUNK, _EMB), jnp.float32)]
        + [pltpu.SemaphoreType.DMA] * (2 * _NBUF + 1)
    ),
)
def _lookup(h_hbm, r_hbm, t_hbm, emb_e_hbm, emb_r_hbm,
            out_h, out_r, out_t, idx_v, rows_v, emb_r_sp, r_out_sp, *sems):
    gsem, ssem, rsem = sems[:_NBUF], sems[_NBUF:2 * _NBUF], sems[2 * _NBUF]
    sid = lax.axis_index("s")
    wid = sid * _NC + lax.axis_index("c")
    c0 = wid * _CPW
    b0 = wid * _BPW

    @pl.when(sid == 0)
    def _stage():
        pltpu.sync_copy(emb_r_hbm, emb_r_sp)

    pltpu.sync_copy(h_hbm.at[pl.ds(c0, _CPW)], idx_v.at[pl.ds(0, _CPW)])
    pltpu.sync_copy(r_hbm.at[pl.ds(c0, _CPW)], idx_v.at[pl.ds(_CPW, _CPW)])
    pltpu.sync_copy(t_hbm.at[pl.ds(c0, _CPW)], idx_v.at[pl.ds(2 * _CPW, _CPW)])

    plsc.subcore_barrier()

    def h_task(j):
        return (j, emb_e_hbm, out_h, (c0 + j) * _CHUNK, False)

    def r_task(j):
        if j < _RSTG:
            return (_CPW + j, emb_r_sp, r_out_sp,
                    (sid * _RSTG + j) * _CHUNK, True)
        return (_CPW + j, emb_r_sp, out_r, (c0 + j) * _CHUNK, False)

    def t_task(j):
        return (2 * _CPW + j, emb_e_hbm, out_t, (c0 + j) * _CHUNK, False)

    tasks = [h_task(0), r_task(0), r_task(1), t_task(0), r_task(2),
             r_task(3), h_task(1), t_task(1), h_task(2), t_task(2),
             h_task(3), t_task(3)]
    last_r = max(i for i, tk in enumerate(tasks) if tk[4])

    def fire_gather(i):
        slot, table, _, _, _ = tasks[i]
        return pltpu.async_copy(
            table.at[idx_v.at[slot]], rows_v.at[i % _NBUF], gsem[i % _NBUF])

    g_desc = [None] * _NTASK
    s_desc = [None] * _NTASK
    r_bulk = None
    for i in range(_LOOK):
        g_desc[i] = fire_gather(i)
    for i in range(_NTASK):
        j = i + _LOOK
        if j < _NTASK:
            if j >= _NBUF:
                if s_desc[j - _NBUF] is not None:
                    s_desc[j - _NBUF].wait()
                    s_desc[j - _NBUF] = None
            g_desc[j] = fire_gather(j)
        _, _, dst, obase, _ = tasks[i]
        g_desc[i].wait()
        s_desc[i] = pltpu.async_copy(
            rows_v.at[i % _NBUF], dst.at[pl.ds(obase, _CHUNK)], ssem[i % _NBUF])
        if i == last_r:
            for k, tk in enumerate(tasks[:i + 1]):
                if tk[4] and s_desc[k] is not None:
                    s_desc[k].wait()
                    s_desc[k] = None
            r_bulk = pltpu.async_copy(
                r_out_sp.at[pl.ds(sid * _RSTG * _CHUNK, _RSTG * _CHUNK)],
                out_r.at[pl.ds(b0, _RSTG * _CHUNK)], rsem)
    for i in range(_NTASK - _NBUF, _NTASK):
        if s_desc[i] is not None:
            s_desc[i].wait()
    r_bulk.wait()


def kernel(x, emb_e, emb_r):
    h = x[:, 0].reshape(_NROWS, _CHUNK)
    r = x[:, 1].reshape(_NROWS, _CHUNK)
    t = x[:, 2].reshape(_NROWS, _CHUNK)
    return _lookup(h, r, t, emb_e, emb_r)


# speedup vs baseline: 1.7518x; 1.0404x over previous
import functools

import jax
import jax.numpy as jnp
from jax import lax
from jax.experimental import pallas as pl
from jax.experimental.pallas import tpu as pltpu
from jax.experimental.pallas import tpu_sc as plsc

_BS = 16384
_EMB = 128
_R_VOCAB = 1000
_CHUNK = 128
_NC = 2
_NS = 16
_NW = _NC * _NS
_NROWS = _BS // _CHUNK
_CPW = _NROWS // _NW
_NTASK = 3 * _CPW
_NBUF = 7
_LOOK = 3

_mesh = plsc.VectorSubcoreMesh(core_axis_name="c", subcore_axis_name="s")


@functools.partial(
    pl.kernel,
    mesh=_mesh,
    out_type=(
        jax.ShapeDtypeStruct((_BS, _EMB), jnp.float32),
        jax.ShapeDtypeStruct((_BS, _EMB), jnp.float32),
        jax.ShapeDtypeStruct((_BS, _EMB), jnp.float32),
    ),
    scratch_types=(
        [pltpu.VMEM((_NTASK, _CHUNK), jnp.int32),
         pltpu.VMEM((_NBUF, _CHUNK, _EMB), jnp.float32),
         pltpu.VMEM_SHARED((_R_VOCAB, _EMB), jnp.float32)]
        + [pltpu.SemaphoreType.DMA] * (2 * _NBUF)
    ),
)
def _lookup(h_hbm, r_hbm, t_hbm, emb_e_hbm, emb_r_hbm,
            out_h, out_r, out_t, idx_v, rows_v, emb_r_sp, *sems):
    gsem, ssem = sems[:_NBUF], sems[_NBUF:]
    wid = lax.axis_index("s") * _NC + lax.axis_index("c")
    c0 = wid * _CPW

    @pl.when(lax.axis_index("s") == 0)
    def _stage():
        pltpu.sync_copy(emb_r_hbm, emb_r_sp)

    pltpu.sync_copy(h_hbm.at[pl.ds(c0, _CPW)], idx_v.at[pl.ds(0, _CPW)])
    pltpu.sync_copy(r_hbm.at[pl.ds(c0, _CPW)], idx_v.at[pl.ds(_CPW, _CPW)])
    pltpu.sync_copy(t_hbm.at[pl.ds(c0, _CPW)], idx_v.at[pl.ds(2 * _CPW, _CPW)])

    plsc.subcore_barrier()

    tasks = []
    for j in range(_CPW):
        tasks.append((j, emb_e_hbm, out_h, (c0 + j) * _CHUNK))
        tasks.append((_CPW + j, emb_r_sp, out_r, (c0 + j) * _CHUNK))
        tasks.append((2 * _CPW + j, emb_e_hbm, out_t, (c0 + j) * _CHUNK))

    def fire_gather(i):
        slot, table, _, _ = tasks[i]
        return pltpu.async_copy(
            table.at[idx_v.at[slot]], rows_v.at[i % _NBUF], gsem[i % _NBUF])

    g_desc = [None] * _NTASK
    s_desc = [None] * _NTASK
    for i in range(_LOOK):
        g_desc[i] = fire_gather(i)
    for i in range(_NTASK):
        j = i + _LOOK
        if j < _NTASK:
            if j >= _NBUF:
                s_desc[j - _NBUF].wait()
            g_desc[j] = fire_gather(j)
        _, _, out, obase = tasks[i]
        g_desc[i].wait()
        s_desc[i] = pltpu.async_copy(
            rows_v.at[i % _NBUF], out.at[pl.ds(obase, _CHUNK)], ssem[i % _NBUF])
    for i in range(_NTASK - _NBUF, _NTASK):
        s_desc[i].wait()


def kernel(x, emb_e, emb_r):
    h = x[:, 0].reshape(_NROWS, _CHUNK)
    r = x[:, 1].reshape(_NROWS, _CHUNK)
    t = x[:, 2].reshape(_NROWS, _CHUNK)
    return _lookup(h, r, t, emb_e, emb_r)


# speedup vs baseline: 1.7609x; 1.0052x over previous
import functools

import jax
import jax.numpy as jnp
from jax import lax
from jax.experimental import pallas as pl
from jax.experimental.pallas import tpu as pltpu
from jax.experimental.pallas import tpu_sc as plsc

_BS = 16384
_EMB = 128
_R_VOCAB = 1000
_CHUNK = 128
_NC = 2
_NS = 16
_NW = _NC * _NS
_NROWS = _BS // _CHUNK
_CPW = _NROWS // _NW
_NTASK = 3 * _CPW
_NBUF = 7
_LOOK = 4

_mesh = plsc.VectorSubcoreMesh(core_axis_name="c", subcore_axis_name="s")


@functools.partial(
    pl.kernel,
    mesh=_mesh,
    out_type=(
        jax.ShapeDtypeStruct((_BS, _EMB), jnp.float32),
        jax.ShapeDtypeStruct((_BS, _EMB), jnp.float32),
        jax.ShapeDtypeStruct((_BS, _EMB), jnp.float32),
    ),
    scratch_types=(
        [pltpu.VMEM((_NTASK, _CHUNK), jnp.int32),
         pltpu.VMEM((_NBUF, _CHUNK, _EMB), jnp.float32),
         pltpu.VMEM_SHARED((_R_VOCAB, _EMB), jnp.float32)]
        + [pltpu.SemaphoreType.DMA] * (2 * _NBUF)
    ),
)
def _lookup(h_hbm, r_hbm, t_hbm, emb_e_hbm, emb_r_hbm,
            out_h, out_r, out_t, idx_v, rows_v, emb_r_sp, *sems):
    gsem, ssem = sems[:_NBUF], sems[_NBUF:]
    wid = lax.axis_index("s") * _NC + lax.axis_index("c")
    c0 = wid * _CPW

    @pl.when(lax.axis_index("s") == 0)
    def _stage():
        pltpu.sync_copy(emb_r_hbm, emb_r_sp)

    pltpu.sync_copy(h_hbm.at[pl.ds(c0, _CPW)], idx_v.at[pl.ds(0, _CPW)])
    pltpu.sync_copy(r_hbm.at[pl.ds(c0, _CPW)], idx_v.at[pl.ds(_CPW, _CPW)])
    pltpu.sync_copy(t_hbm.at[pl.ds(c0, _CPW)], idx_v.at[pl.ds(2 * _CPW, _CPW)])

    plsc.subcore_barrier()

    tasks = []
    for j in range(_CPW):
        tasks.append((j, emb_e_hbm, out_h, (c0 + j) * _CHUNK))
        tasks.append((_CPW + j, emb_r_sp, out_r, (c0 + j) * _CHUNK))
        tasks.append((2 * _CPW + j, emb_e_hbm, out_t, (c0 + j) * _CHUNK))

    def fire_gather(i):
        slot, table, _, _ = tasks[i]
        return pltpu.async_copy(
            table.at[idx_v.at[slot]], rows_v.at[i % _NBUF], gsem[i % _NBUF])

    g_desc = [None] * _NTASK
    s_desc = [None] * _NTASK
    for i in range(_LOOK):
        g_desc[i] = fire_gather(i)
    for i in range(_NTASK):
        j = i + _LOOK
        if j < _NTASK:
            if j >= _NBUF:
                s_desc[j - _NBUF].wait()
            g_desc[j] = fire_gather(j)
        _, _, out, obase = tasks[i]
        g_desc[i].wait()
        s_desc[i] = pltpu.async_copy(
            rows_v.at[i % _NBUF], out.at[pl.ds(obase, _CHUNK)], ssem[i % _NBUF])
    for i in range(_NTASK - _NBUF, _NTASK):
        s_desc[i].wait()


def kernel(x, emb_e, emb_r):
    h = x[:, 0].reshape(_NROWS, _CHUNK)
    r = x[:, 1].reshape(_NROWS, _CHUNK)
    t = x[:, 2].reshape(_NROWS, _CHUNK)
    return _lookup(h, r, t, emb_e, emb_r)
